# baseline (device time: 17049 ns/iter reference)
import jax
import jax.numpy as jnp
from jax import lax
from jax.experimental import pallas as pl
from jax.experimental.pallas import tpu as pltpu

N_CHUNK = 4


def kernel(x):
    x = x.astype(jnp.bfloat16)
    m, n = x.shape
    half = n // 2
    rh = m // 2
    rpc = rh // N_CHUNK

    def body(x_ref, out_ref, y_send, y_recv, x_send, x_recv, loc_sem):
        my_x = lax.axis_index("x")
        my_y = lax.axis_index("y")
        peer_y = (my_x, 1 - my_y)
        peer_x = (1 - my_x, my_y)

        barrier_sem = pltpu.get_barrier_semaphore()
        for nbr in (peer_y, peer_x):
            pl.semaphore_signal(
                barrier_sem, inc=1,
                device_id=nbr, device_id_type=pl.DeviceIdType.MESH,
            )
        pl.semaphore_wait(barrier_sem, 2)

        send_rows = my_x * rh
        dst_base = my_y * m + my_x * rh
        yrecv_base = (1 - my_y) * m + my_x * rh
        xrecv_base = (1 - my_y) * m + (1 - my_x) * rh

        for c in range(N_CHUNK):
            lo = c * rpc

            @pl.when(my_y == 0)
            def _(lo=lo):
                pltpu.make_async_remote_copy(
                    src_ref=x_ref.at[
                        pl.ds(send_rows + lo, rpc), pl.ds(half, half)
                    ],
                    dst_ref=out_ref.at[pl.ds(dst_base + lo, rpc), :],
                    send_sem=y_send.at[c],
                    recv_sem=y_recv.at[c],
                    device_id=peer_y,
                    device_id_type=pl.DeviceIdType.MESH,
                ).start()

            @pl.when(my_y == 1)
            def _(lo=lo):
                pltpu.make_async_remote_copy(
                    src_ref=x_ref.at[
                        pl.ds(send_rows + lo, rpc), pl.ds(0, half)
                    ],
                    dst_ref=out_ref.at[pl.ds(dst_base + lo, rpc), :],
                    send_sem=y_send.at[c],
                    recv_sem=y_recv.at[c],
                    device_id=peer_y,
                    device_id_type=pl.DeviceIdType.MESH,
                ).start()

        @pl.when(my_y == 0)
        def _():
            pltpu.make_async_copy(
                x_ref.at[:, pl.ds(0, half)],
                out_ref.at[pl.ds(0, m), :],
                loc_sem,
            ).start()

        @pl.when(my_y == 1)
        def _():
            pltpu.make_async_copy(
                x_ref.at[:, pl.ds(half, half)],
                out_ref.at[pl.ds(m, m), :],
                loc_sem,
            ).start()

        x_rdmas = []
        for c in range(N_CHUNK):
            lo = c * rpc
            recv = pltpu.make_async_remote_copy(
                src_ref=out_ref.at[pl.ds(yrecv_base + lo, rpc), :],
                dst_ref=out_ref.at[pl.ds(yrecv_base + lo, rpc), :],
                send_sem=y_send.at[c],
                recv_sem=y_recv.at[c],
                device_id=peer_y,
                device_id_type=pl.DeviceIdType.MESH,
            )
            recv.wait_recv()
            fwd = pltpu.make_async_remote_copy(
                src_ref=out_ref.at[pl.ds(yrecv_base + lo, rpc), :],
                dst_ref=out_ref.at[pl.ds(yrecv_base + lo, rpc), :],
                send_sem=x_send.at[c],
                recv_sem=x_recv.at[c],
                device_id=peer_x,
                device_id_type=pl.DeviceIdType.MESH,
            )
            fwd.start()
            x_rdmas.append(fwd)

        for c in range(N_CHUNK):
            lo = c * rpc
            pltpu.make_async_remote_copy(
                src_ref=out_ref.at[pl.ds(xrecv_base + lo, rpc), :],
                dst_ref=out_ref.at[pl.ds(xrecv_base + lo, rpc), :],
                send_sem=x_send.at[c],
                recv_sem=x_recv.at[c],
                device_id=peer_x,
                device_id_type=pl.DeviceIdType.MESH,
            ).wait_recv()

        for c in range(N_CHUNK):
            lo = c * rpc
            pltpu.make_async_remote_copy(
                src_ref=x_ref.at[pl.ds(send_rows + lo, rpc), pl.ds(0, half)],
                dst_ref=out_ref.at[pl.ds(dst_base + lo, rpc), :],
                send_sem=y_send.at[c],
                recv_sem=y_recv.at[c],
                device_id=peer_y,
                device_id_type=pl.DeviceIdType.MESH,
            ).wait_send()
        for r in x_rdmas:
            r.wait_send()

        @pl.when(my_y == 0)
        def _():
            pltpu.make_async_copy(
                x_ref.at[:, pl.ds(0, half)],
                out_ref.at[pl.ds(0, m), :],
                loc_sem,
            ).wait()

        @pl.when(my_y == 1)
        def _():
            pltpu.make_async_copy(
                x_ref.at[:, pl.ds(half, half)],
                out_ref.at[pl.ds(m, m), :],
                loc_sem,
            ).wait()

    return pl.pallas_call(
        body,
        out_shape=jax.ShapeDtypeStruct((2 * m, half), jnp.bfloat16),
        in_specs=[pl.BlockSpec(memory_space=pltpu.VMEM)],
        out_specs=pl.BlockSpec(memory_space=pltpu.VMEM),
        scratch_shapes=[
            pltpu.SemaphoreType.DMA((N_CHUNK,)),
            pltpu.SemaphoreType.DMA((N_CHUNK,)),
            pltpu.SemaphoreType.DMA((N_CHUNK,)),
            pltpu.SemaphoreType.DMA((N_CHUNK,)),
            pltpu.SemaphoreType.DMA,
        ],
        compiler_params=pltpu.CompilerParams(collective_id=0),
    )(x)


# device time: 16271 ns/iter; 1.0478x vs baseline; 1.0478x over previous
import jax
import jax.numpy as jnp
from jax import lax
from jax.experimental import pallas as pl
from jax.experimental.pallas import tpu as pltpu

CHUNK_ROWS = (64,) * 8
N_CHUNK = len(CHUNK_ROWS)
CHUNK_LO = tuple(sum(CHUNK_ROWS[:i]) for i in range(N_CHUNK))


def kernel(x):
    m, n = x.shape
    half = n // 2
    rh = m // 2
    assert sum(CHUNK_ROWS) == rh
    bf16 = jnp.bfloat16

    def body(x_ref, out_ref, send_buf, y_send, y_recv, x_send, x_recv):
        my_x = lax.axis_index("x")
        my_y = lax.axis_index("y")
        peer_y = (my_x, 1 - my_y)
        peer_x = (1 - my_x, my_y)

        for xv in (0, 1):
            for yv in (0, 1):

                @pl.when((my_x == xv) & (my_y == yv))
                def _(xv=xv, yv=yv):
                    send_buf[...] = x_ref[
                        xv * rh:(xv + 1) * rh,
                        (1 - yv) * half:(2 - yv) * half,
                    ].astype(bf16)

        barrier_sem = pltpu.get_barrier_semaphore()
        for nbr in (peer_y, peer_x):
            pl.semaphore_signal(
                barrier_sem, inc=1,
                device_id=nbr, device_id_type=pl.DeviceIdType.MESH,
            )
        pl.semaphore_wait(barrier_sem, 2)

        dst_base = my_y * m + my_x * rh
        yrecv_base = (1 - my_y) * m + my_x * rh
        xrecv_base = (1 - my_y) * m + (1 - my_x) * rh

        y_rdmas = []
        for c in range(N_CHUNK):
            lo, nr = CHUNK_LO[c], CHUNK_ROWS[c]
            rdma = pltpu.make_async_remote_copy(
                src_ref=send_buf.at[pl.ds(lo, nr), :],
                dst_ref=out_ref.at[pl.ds(dst_base + lo, nr), :],
                send_sem=y_send.at[c],
                recv_sem=y_recv.at[c],
                device_id=peer_y,
                device_id_type=pl.DeviceIdType.MESH,
            )
            rdma.start()
            y_rdmas.append(rdma)

        @pl.when(my_y == 0)
        def _():
            out_ref[pl.ds(0, m), :] = x_ref[:, 0:half].astype(bf16)

        @pl.when(my_y == 1)
        def _():
            out_ref[pl.ds(m, m), :] = x_ref[:, half:n].astype(bf16)

        x_rdmas = []
        for c in range(N_CHUNK):
            lo, nr = CHUNK_LO[c], CHUNK_ROWS[c]
            recv = pltpu.make_async_remote_copy(
                src_ref=send_buf.at[pl.ds(lo, nr), :],
                dst_ref=out_ref.at[pl.ds(yrecv_base + lo, nr), :],
                send_sem=y_send.at[c],
                recv_sem=y_recv.at[c],
                device_id=peer_y,
                device_id_type=pl.DeviceIdType.MESH,
            )
            recv.wait_recv()
            fwd = pltpu.make_async_remote_copy(
                src_ref=out_ref.at[pl.ds(yrecv_base + lo, nr), :],
                dst_ref=out_ref.at[pl.ds(yrecv_base + lo, nr), :],
                send_sem=x_send.at[c],
                recv_sem=x_recv.at[c],
                device_id=peer_x,
                device_id_type=pl.DeviceIdType.MESH,
            )
            fwd.start()
            x_rdmas.append(fwd)

        for c in range(N_CHUNK):
            lo, nr = CHUNK_LO[c], CHUNK_ROWS[c]
            pltpu.make_async_remote_copy(
                src_ref=send_buf.at[pl.ds(lo, nr), :],
                dst_ref=out_ref.at[pl.ds(xrecv_base + lo, nr), :],
                send_sem=x_send.at[c],
                recv_sem=x_recv.at[c],
                device_id=peer_x,
                device_id_type=pl.DeviceIdType.MESH,
            ).wait_recv()

        for r in y_rdmas:
            r.wait_send()
        for r in x_rdmas:
            r.wait_send()

    return pl.pallas_call(
        body,
        out_shape=jax.ShapeDtypeStruct((2 * m, half), bf16),
        in_specs=[pl.BlockSpec(memory_space=pltpu.VMEM)],
        out_specs=pl.BlockSpec(memory_space=pltpu.VMEM),
        scratch_shapes=[
            pltpu.VMEM((rh, half), bf16),
            pltpu.SemaphoreType.DMA((N_CHUNK,)),
            pltpu.SemaphoreType.DMA((N_CHUNK,)),
            pltpu.SemaphoreType.DMA((N_CHUNK,)),
            pltpu.SemaphoreType.DMA((N_CHUNK,)),
        ],
        compiler_params=pltpu.CompilerParams(collective_id=0),
    )(x)
